# Initial kernel scaffold; baseline (speedup 1.0000x reference)
import jax
import jax.numpy as jnp
from jax.experimental import pallas as pl


def kernel(z, codebook):
    raise NotImplementedError("write your pallas kernel here")


# speedup vs baseline: 1.4102x; 1.4102x over previous
import functools

import jax
import jax.numpy as jnp
from jax import lax
from jax.experimental import pallas as pl
from jax.experimental.pallas import tpu as pltpu
from jax.experimental.pallas import tpu_sc as plsc

K = 8192
D = 32
N = 16384

ROWS = 512
NB = N // ROWS

NC = 2
NS = 16
NW = NC * NS
BPW = N // NW


NCHUNK = 2
CS = K // NCHUNK


def _tc_body(z_ref, cbt_ref, idx_ref, loss_ref):
    i = pl.program_id(0)
    z = z_ref[...]
    cbt = cbt_ref[...]
    zm2 = (z + z).astype(jnp.bfloat16)
    cbt_bf = cbt.astype(jnp.bfloat16)
    mm2 = lax.dot_general(zm2, cbt_bf, (((1,), (0,)), ((), ())),
                          preferred_element_type=jnp.float32)
    z2 = jnp.sum(z * z, axis=1, keepdims=True)
    e2 = jnp.sum(cbt * cbt, axis=0, keepdims=True)
    dist = (z2 - mm2) + e2

    acc = jnp.full((ROWS,), jnp.inf, jnp.float32)
    vex = jnp.zeros((ROWS,), jnp.float32)
    idx = jnp.zeros((ROWS,), jnp.int32)
    for c in range(NCHUNK):
        sl = dist[:, c*CS:(c+1)*CS]
        vm = jnp.min(sl, axis=1)
        im = (jnp.argmin(sl, axis=1) + c*CS).astype(jnp.int32)
        take = vm < acc
        idx = jnp.where(take, im, idx)
        vex = jnp.where(take, vm, vex)
        acc = jnp.where(take, vm, acc).astype(jnp.bfloat16).astype(jnp.float32)
    idx_ref[0, 0, :] = idx

    part = jnp.sum(vex).reshape(1, 1)

    @pl.when(i == 0)
    def _():
        loss_ref[...] = jnp.zeros((1, 1), jnp.float32)

    loss_ref[...] += part


def _tc_argmin(zflat, cbt):
    return pl.pallas_call(
        _tc_body,
        grid=(NB,),
        in_specs=[
            pl.BlockSpec((ROWS, D), lambda i: (i, 0)),
            pl.BlockSpec((D, K), lambda i: (0, 0)),
        ],
        out_specs=[
            pl.BlockSpec((1, 1, ROWS), lambda i: (i, 0, 0)),
            pl.BlockSpec((1, 1), lambda i: (0, 0)),
        ],
        out_shape=[
            jax.ShapeDtypeStruct((NB, 1, ROWS), jnp.int32),
            jax.ShapeDtypeStruct((1, 1), jnp.float32),
        ],
    )(zflat, cbt)


DP = 128


@functools.cache
def _sc_gather_fn():
    mesh = plsc.VectorSubcoreMesh(core_axis_name="c", subcore_axis_name="s")

    @functools.partial(
        pl.kernel,
        mesh=mesh,
        out_type=jax.ShapeDtypeStruct((N, DP), jnp.float32),
        scratch_types=[
            pltpu.VMEM((BPW,), jnp.int32),
            pltpu.VMEM((BPW, DP), jnp.float32),
            pltpu.SemaphoreType.DMA,
        ],
    )
    def _sc_gather(table_hbm, idx_hbm, out_hbm, idx_v, rows_v, sem):
        wid = lax.axis_index("s") * NC + lax.axis_index("c")
        base = wid * BPW
        pltpu.sync_copy(idx_hbm.at[pl.ds(base, BPW)], idx_v)
        pltpu.async_copy(table_hbm.at[idx_v], rows_v, sem).wait()
        pltpu.sync_copy(rows_v, out_hbm.at[pl.ds(base, BPW)])

    return _sc_gather


def kernel(z, codebook):
    b, t, d = z.shape
    zflat = z.reshape(-1, d)
    cbt = codebook.T

    idx_blk, loss_acc = _tc_argmin(zflat, cbt)
    idx_flat = idx_blk.reshape(-1)

    cb_pad = jnp.pad(codebook, ((0, 0), (0, DP - D)))
    z_q = _sc_gather_fn()(cb_pad, idx_flat)[:, :D]

    loss = loss_acc[0, 0] / (N * D)
    z_q_st = z_q.reshape(b, t, d)
    return z_q_st, idx_flat.reshape(b, t), loss, loss
